# baseline (speedup 1.0000x reference)
import jax
import jax.numpy as jnp
from jax.experimental import pallas as pl
from jax.experimental.pallas import tpu as pltpu

_H = 32
_W = 32
_N = _H * _W
_B = 8
_C = 192


def _vshift(a, dy, lane):
    if dy == 0:
        return a
    if dy == 1:
        return jnp.where(lane >= _N - _W, a, jnp.roll(a, -_W, axis=1))
    return jnp.where(lane < _W, a, jnp.roll(a, _W, axis=1))


def _hshift(a, dx, xcol):
    if dx == 0:
        return a
    if dx == 1:
        return jnp.where(xcol == _W - 1, a, jnp.roll(a, -1, axis=1))
    return jnp.where(xcol == 0, a, jnp.roll(a, 1, axis=1))


def _normalize_cols(v):
    n = jnp.sqrt(jnp.sum(v * v, axis=0, keepdims=True))
    return v / jnp.maximum(n, 1e-12)


def _body(x_ref, w_ref, bnw_ref, bnb_ref, y_ref, pre_scr, s1_scr, s2_scr):
    i = pl.program_id(0)

    @pl.when(i < _B)
    def _phase1():
        prev = x_ref[0, 0].reshape(_C, _N)
        nxt = x_ref[0, 1].reshape(_C, _N)
        cur = x_ref[0, 2].reshape(_C, _N)

        lane = jax.lax.broadcasted_iota(jnp.int32, (1, _N), 1)
        xcol = lane & (_W - 1)
        xm_m = xcol == 0
        xm_p = xcol == _W - 1
        ym_m = lane < _W
        ym_p = lane >= _N - _W
        cur_n = _normalize_cols(cur)

        def find_nn(y):
            yk = _normalize_cols(y)
            hs = {}
            sraw = {}
            for dy in (-1, 0, 1):
                v = yk if dy == 0 else jnp.roll(yk, -_W * dy, axis=1)
                for dx in (-1, 0, 1):
                    m = v if dx == 0 else jnp.roll(v, -dx, axis=1)
                    hs[(dy, dx)] = m
                    sraw[(dy, dx)] = jnp.sum(cur_n * m, axis=0, keepdims=True)
            sx = {}
            for dy in (-1, 0, 1):
                sx[(dy, -1)] = jnp.where(xm_m, sraw[(dy, 0)], sraw[(dy, -1)])
                sx[(dy, 0)] = sraw[(dy, 0)]
                sx[(dy, 1)] = jnp.where(xm_p, sraw[(dy, 0)], sraw[(dy, 1)])
            best = None
            bestd = None
            d = 0
            for dy in (-1, 0, 1):
                for dx in (-1, 0, 1):
                    if dy == -1:
                        s = jnp.where(ym_m, sx[(0, dx)], sx[(-1, dx)])
                    elif dy == 1:
                        s = jnp.where(ym_p, sx[(0, dx)], sx[(1, dx)])
                    else:
                        s = sx[(0, dx)]
                    if best is None:
                        best = s
                        bestd = jnp.zeros((1, _N), jnp.int32)
                    else:
                        better = s < best
                        best = jnp.where(better, s, best)
                        bestd = jnp.where(better, d, bestd)
                    d += 1
            dyq = bestd // 3
            dxq = bestd - dyq * 3
            dyc = jnp.where(ym_m, jnp.maximum(dyq, 1), dyq)
            dyc = jnp.where(ym_p, jnp.minimum(dyc, 1), dyc)
            dxc = jnp.where(xm_m, jnp.maximum(dxq, 1), dxq)
            dxc = jnp.where(xm_p, jnp.minimum(dxc, 1), dxc)
            e = dyc * 3 + dxc
            nn = hs[(-1, -1)]
            d = 0
            for dy in (-1, 0, 1):
                for dx in (-1, 0, 1):
                    if d > 0:
                        nn = jnp.where(e == d, hs[(dy, dx)], nn)
                    d += 1
            return nn

        nn_prev = find_nn(prev)
        nn_next = find_nn(nxt)
        cat = jnp.concatenate([nn_prev, nn_next, cur], axis=0)
        out = jax.lax.dot_general(
            w_ref[...], cat, (((1,), (0,)), ((), ())),
            preferred_element_type=jnp.float32,
            precision=jax.lax.Precision.HIGHEST)
        pre_scr[i] = out
        p1 = jnp.sum(out, axis=1, keepdims=True)
        p2 = jnp.sum(out * out, axis=1, keepdims=True)
        is0 = i == 0
        s1_scr[...] = jnp.where(is0, p1, s1_scr[...] + p1)
        s2_scr[...] = jnp.where(is0, p2, s2_scr[...] + p2)

    @pl.when(i >= _B)
    def _phase2():
        b = i - _B
        cnt = float(_B * _N)
        mean = s1_scr[...] / cnt
        var = s2_scr[...] / cnt - mean * mean
        inv = bnw_ref[...] / jnp.sqrt(var + 1e-5)
        y_ref[0, :2] = x_ref[0, :2]
        y = jnp.maximum((pre_scr[b] - mean) * inv + bnb_ref[...], 0.0)
        y_ref[0, 2] = y.reshape(_C, _H, _W)


def kernel(x, conv_w, bn_w, bn_b):
    b, f, c, h, w = x.shape

    y = pl.pallas_call(
        _body,
        grid=(2 * b,),
        in_specs=[
            pl.BlockSpec((1, f, c, h, w),
                         lambda i: (jax.lax.rem(i, _B), 0, 0, 0, 0)),
            pl.BlockSpec((c, f * c), lambda i: (0, 0)),
            pl.BlockSpec((c, 1), lambda i: (0, 0)),
            pl.BlockSpec((c, 1), lambda i: (0, 0)),
        ],
        out_specs=pl.BlockSpec(
            (1, f, c, h, w), lambda i: (jnp.maximum(i - _B, 0), 0, 0, 0, 0)),
        out_shape=jax.ShapeDtypeStruct((b, f, c, h, w), jnp.float32),
        scratch_shapes=[
            pltpu.VMEM((_B, _C, _N), jnp.float32),
            pltpu.VMEM((_C, 1), jnp.float32),
            pltpu.VMEM((_C, 1), jnp.float32),
        ],
    )(x, conv_w, bn_w.reshape(c, 1), bn_b.reshape(c, 1))

    return y


# speedup vs baseline: 1.8521x; 1.8521x over previous
import jax
import jax.numpy as jnp
from jax.experimental import pallas as pl
from jax.experimental.pallas import tpu as pltpu

_H = 32
_W = 32
_N = _H * _W
_B = 8
_C = 192


def _vshift(a, dy, lane):
    if dy == 0:
        return a
    if dy == 1:
        return jnp.where(lane >= _N - _W, a, jnp.roll(a, -_W, axis=1))
    return jnp.where(lane < _W, a, jnp.roll(a, _W, axis=1))


def _hshift(a, dx, xcol):
    if dx == 0:
        return a
    if dx == 1:
        return jnp.where(xcol == _W - 1, a, jnp.roll(a, -1, axis=1))
    return jnp.where(xcol == 0, a, jnp.roll(a, 1, axis=1))


def _normalize_cols(v):
    n = jnp.sqrt(jnp.sum(v * v, axis=0, keepdims=True))
    return v / jnp.maximum(n, 1e-12)


def _body(x_ref, w_ref, bnw_ref, bnb_ref, y_ref, pre_scr, s1_scr, s2_scr):
    i = pl.program_id(0)

    @pl.when(i < _B)
    def _phase1():
        prev = x_ref[0, 0]
        nxt = x_ref[0, 1]
        cur = x_ref[0, 2]

        lane = jax.lax.broadcasted_iota(jnp.int32, (1, _N), 1)
        xcol = lane & (_W - 1)
        xm_m = xcol == 0
        xm_p = xcol == _W - 1
        ym_m = lane < _W
        ym_p = lane >= _N - _W
        cur_n = _normalize_cols(cur)

        def find_nn(y):
            yk = _normalize_cols(y)
            hs = {}
            sraw = {}
            for dy in (-1, 0, 1):
                v = yk if dy == 0 else jnp.roll(yk, -_W * dy, axis=1)
                for dx in (-1, 0, 1):
                    m = v if dx == 0 else jnp.roll(v, -dx, axis=1)
                    hs[(dy, dx)] = m
                    sraw[(dy, dx)] = jnp.sum(cur_n * m, axis=0, keepdims=True)
            sx = {}
            for dy in (-1, 0, 1):
                sx[(dy, -1)] = jnp.where(xm_m, sraw[(dy, 0)], sraw[(dy, -1)])
                sx[(dy, 0)] = sraw[(dy, 0)]
                sx[(dy, 1)] = jnp.where(xm_p, sraw[(dy, 0)], sraw[(dy, 1)])
            best = None
            bestd = None
            d = 0
            for dy in (-1, 0, 1):
                for dx in (-1, 0, 1):
                    if dy == -1:
                        s = jnp.where(ym_m, sx[(0, dx)], sx[(-1, dx)])
                    elif dy == 1:
                        s = jnp.where(ym_p, sx[(0, dx)], sx[(1, dx)])
                    else:
                        s = sx[(0, dx)]
                    if best is None:
                        best = s
                        bestd = jnp.zeros((1, _N), jnp.int32)
                    else:
                        better = s < best
                        best = jnp.where(better, s, best)
                        bestd = jnp.where(better, d, bestd)
                    d += 1
            dyq = bestd // 3
            dxq = bestd - dyq * 3
            dyc = jnp.where(ym_m, jnp.maximum(dyq, 1), dyq)
            dyc = jnp.where(ym_p, jnp.minimum(dyc, 1), dyc)
            dxc = jnp.where(xm_m, jnp.maximum(dxq, 1), dxq)
            dxc = jnp.where(xm_p, jnp.minimum(dxc, 1), dxc)
            e = dyc * 3 + dxc
            nn = hs[(-1, -1)]
            d = 0
            for dy in (-1, 0, 1):
                for dx in (-1, 0, 1):
                    if d > 0:
                        nn = jnp.where(e == d, hs[(dy, dx)], nn)
                    d += 1
            return nn

        nn_prev = find_nn(prev)
        nn_next = find_nn(nxt)
        cat = jnp.concatenate([nn_prev, nn_next, cur], axis=0)
        out = jax.lax.dot_general(
            w_ref[...], cat, (((1,), (0,)), ((), ())),
            preferred_element_type=jnp.float32,
            precision=jax.lax.Precision.HIGHEST)
        pre_scr[i] = out
        p1 = jnp.sum(out, axis=1, keepdims=True)
        p2 = jnp.sum(out * out, axis=1, keepdims=True)
        is0 = i == 0
        s1_scr[...] = jnp.where(is0, p1, s1_scr[...] + p1)
        s2_scr[...] = jnp.where(is0, p2, s2_scr[...] + p2)

    @pl.when(i == _B)
    def _phase2():
        cnt = float(_B * _N)
        mean = s1_scr[...] / cnt
        var = s2_scr[...] / cnt - mean * mean
        inv = bnw_ref[...] / jnp.sqrt(var + 1e-5)
        y_ref[...] = jnp.maximum(
            (pre_scr[...] - mean[None]) * inv[None] + bnb_ref[...][None], 0.0)


def kernel(x, conv_w, bn_w, bn_b):
    b, f, c, h, w = x.shape
    n = h * w
    x4 = x.reshape(b, f, c, n)

    y = pl.pallas_call(
        _body,
        grid=(b + 1,),
        in_specs=[
            pl.BlockSpec((1, f, c, n), lambda i: (jnp.minimum(i, _B - 1), 0, 0, 0)),
            pl.BlockSpec((c, f * c), lambda i: (0, 0)),
            pl.BlockSpec((c, 1), lambda i: (0, 0)),
            pl.BlockSpec((c, 1), lambda i: (0, 0)),
        ],
        out_specs=pl.BlockSpec((b, c, n), lambda i: (0, 0, 0)),
        out_shape=jax.ShapeDtypeStruct((b, c, n), jnp.float32),
        scratch_shapes=[
            pltpu.VMEM((_B, _C, _N), jnp.float32),
            pltpu.VMEM((_C, 1), jnp.float32),
            pltpu.VMEM((_C, 1), jnp.float32),
        ],
    )(x4, conv_w, bn_w.reshape(c, 1), bn_b.reshape(c, 1))

    return jnp.stack([x[:, 0], x[:, 1], y.reshape(b, c, h, w)], axis=1)


# speedup vs baseline: 2.7462x; 1.4828x over previous
import jax
import jax.numpy as jnp
from jax.experimental import pallas as pl
from jax.experimental.pallas import tpu as pltpu

_H = 32
_W = 32
_N = _H * _W
_B = 8
_C = 192
_POFF = 40


def _normalize_rows(v):
    n = jnp.sqrt(jnp.sum(v * v, axis=1, keepdims=True))
    return v / jnp.maximum(n, 1e-12)


def _body(x_ref, w1_ref, w2_ref, w3_ref, bnw_ref, bnb_ref, y_ref,
          pre_scr, p_scr, s1_scr, s2_scr):
    i = pl.program_id(0)

    @pl.when(i < _B)
    def _phase1():
        prev = x_ref[0, 0]
        nxt = x_ref[0, 1]
        cur = x_ref[0, 2]

        p = jax.lax.broadcasted_iota(jnp.int32, (_N, 1), 0)
        xcol = p & (_W - 1)
        xm_m = xcol == 0
        xm_p = xcol == _W - 1
        cur_n = _normalize_rows(cur)

        def find_nn(y):
            yk = _normalize_rows(y)
            p_scr[pl.ds(8, _W), :] = yk[0:_W]
            p_scr[pl.ds(_POFF, _N), :] = yk
            p_scr[pl.ds(_POFF + _N, _W), :] = yk[_N - _W:_N]
            hs = {}
            sraw = {}
            for dy in (-1, 0, 1):
                for dx in (-1, 0, 1):
                    m = p_scr[pl.ds(_POFF + dy * _W + dx, _N), :]
                    hs[(dy, dx)] = m
                    sraw[(dy, dx)] = jnp.sum(cur_n * m, axis=1, keepdims=True)
            best = None
            bestd = None
            d = 0
            for dy in (-1, 0, 1):
                for dx in (-1, 0, 1):
                    if dx == -1:
                        s = jnp.where(xm_m, sraw[(dy, 0)], sraw[(dy, -1)])
                    elif dx == 1:
                        s = jnp.where(xm_p, sraw[(dy, 0)], sraw[(dy, 1)])
                    else:
                        s = sraw[(dy, 0)]
                    if best is None:
                        best = s
                        bestd = jnp.zeros((_N, 1), jnp.int32)
                    else:
                        better = s < best
                        best = jnp.where(better, s, best)
                        bestd = jnp.where(better, d, bestd)
                    d += 1
            dyq = bestd // 3
            dxq = bestd - dyq * 3
            dxc = jnp.where(xm_m, jnp.maximum(dxq, 1), dxq)
            dxc = jnp.where(xm_p, jnp.minimum(dxc, 1), dxc)
            e = dyq * 3 + dxc
            nn = hs[(-1, -1)]
            d = 0
            for dy in (-1, 0, 1):
                for dx in (-1, 0, 1):
                    if d > 0:
                        nn = jnp.where(e == d, hs[(dy, dx)], nn)
                    d += 1
            return nn

        nn_prev = find_nn(prev)
        nn_next = find_nn(nxt)
        dn = (((1,), (1,)), ((), ()))
        out = jax.lax.dot_general(
            nn_prev, w1_ref[...], dn, preferred_element_type=jnp.float32,
            precision=jax.lax.Precision.HIGHEST)
        out += jax.lax.dot_general(
            nn_next, w2_ref[...], dn, preferred_element_type=jnp.float32,
            precision=jax.lax.Precision.HIGHEST)
        out += jax.lax.dot_general(
            cur, w3_ref[...], dn, preferred_element_type=jnp.float32,
            precision=jax.lax.Precision.HIGHEST)
        pre_scr[i] = out
        p1 = jnp.sum(out, axis=0, keepdims=True)
        p2 = jnp.sum(out * out, axis=0, keepdims=True)
        is0 = i == 0
        s1_scr[...] = jnp.where(is0, p1, s1_scr[...] + p1)
        s2_scr[...] = jnp.where(is0, p2, s2_scr[...] + p2)

    @pl.when(i >= _B)
    def _phase2():
        b = i - _B
        cnt = float(_B * _N)
        mean = s1_scr[...] / cnt
        var = s2_scr[...] / cnt - mean * mean
        inv = bnw_ref[...] / jnp.sqrt(var + 1e-5)
        y_ref[0, 0] = x_ref[0, 0]
        y_ref[0, 1] = x_ref[0, 1]
        y_ref[0, 2] = jnp.maximum((pre_scr[b] - mean) * inv + bnb_ref[...], 0.0)


def kernel(x, conv_w, bn_w, bn_b):
    b, f, c, h, w = x.shape
    n = h * w
    xt = x.transpose(0, 1, 3, 4, 2).reshape(b, f, n, c)
    w1 = conv_w[:, 0:c]
    w2 = conv_w[:, c:2 * c]
    w3 = conv_w[:, 2 * c:3 * c]

    y = pl.pallas_call(
        _body,
        grid=(2 * b,),
        in_specs=[
            pl.BlockSpec((1, f, n, c), lambda i: (jax.lax.rem(i, _B), 0, 0, 0)),
            pl.BlockSpec((c, c), lambda i: (0, 0)),
            pl.BlockSpec((c, c), lambda i: (0, 0)),
            pl.BlockSpec((c, c), lambda i: (0, 0)),
            pl.BlockSpec((1, c), lambda i: (0, 0)),
            pl.BlockSpec((1, c), lambda i: (0, 0)),
        ],
        out_specs=pl.BlockSpec(
            (1, f, n, c), lambda i: (jnp.maximum(i - _B, 0), 0, 0, 0)),
        out_shape=jax.ShapeDtypeStruct((b, f, n, c), jnp.float32),
        scratch_shapes=[
            pltpu.VMEM((_B, _N, _C), jnp.float32),
            pltpu.VMEM((_POFF + _N + _W + 8, _C), jnp.float32),
            pltpu.VMEM((1, _C), jnp.float32),
            pltpu.VMEM((1, _C), jnp.float32),
        ],
    )(xt, w1, w2, w3, bn_w.reshape(1, c), bn_b.reshape(1, c))

    return y.reshape(b, f, h, w, c).transpose(0, 1, 4, 2, 3)
